# Initial kernel scaffold; baseline (speedup 1.0000x reference)
import jax
import jax.numpy as jnp
from jax.experimental import pallas as pl


def kernel(x, w1_t, b1_r, w2_t, b2_r):
    raise NotImplementedError("write your pallas kernel here")


# speedup vs baseline: 1.4799x; 1.4799x over previous
import functools

import jax
import jax.numpy as jnp
from jax.experimental import pallas as pl
from jax.experimental.pallas import tpu as pltpu


def _cdiv(a: int, b: int) -> int:
    return (a + b - 1) // b


def _mlp_kernel(x_ref, w1_ref, b1_ref, w2_ref, b2_ref, o_ref):
    xb = x_ref[...].astype(jnp.bfloat16)
    h = jnp.dot(xb, w1_ref[...], preferred_element_type=jnp.float32)
    h = jnp.maximum(h + b1_ref[...], 0.0).astype(jnp.bfloat16)
    y = jnp.dot(h, w2_ref[...], preferred_element_type=jnp.float32)
    o_ref[...] = (y + b2_ref[...]).astype(o_ref.dtype)


@jax.jit
def kernel(x, w1_t, b1_r, w2_t, b2_r):
    b, n_feature = x.shape
    n_hidden, n_output = w2_t.shape

    w1_bf = w1_t.astype(jnp.bfloat16)
    w2_bf = w2_t.astype(jnp.bfloat16)
    b1_f = b1_r.astype(jnp.float32)
    b2_f = b2_r.astype(jnp.float32)

    tb = min(1024, max(8, _cdiv(b, 8) * 8))
    nb = _cdiv(b, tb)
    b_pad = nb * tb
    if b_pad != b:
        x = jnp.pad(x, ((0, b_pad - b), (0, 0)))

    out = pl.pallas_call(
        _mlp_kernel,
        out_shape=jax.ShapeDtypeStruct((b_pad, n_output), x.dtype),
        grid=(nb,),
        in_specs=[
            pl.BlockSpec((tb, n_feature), lambda i: (i, 0)),
            pl.BlockSpec((n_feature, n_hidden), lambda i: (0, 0)),
            pl.BlockSpec((1, n_hidden), lambda i: (0, 0)),
            pl.BlockSpec((n_hidden, n_output), lambda i: (0, 0)),
            pl.BlockSpec((1, n_output), lambda i: (0, 0)),
        ],
        out_specs=pl.BlockSpec((tb, n_output), lambda i: (i, 0)),
        compiler_params=pltpu.CompilerParams(
            dimension_semantics=("parallel",),
            vmem_limit_bytes=int(64 * 1024 * 1024 * 0.92)),
    )(x, w1_bf, b1_f, w2_bf, b2_f)

    if b_pad != b:
        out = out[:b]
    return out


# speedup vs baseline: 1.6148x; 1.0912x over previous
import functools

import jax
import jax.numpy as jnp
from jax.experimental import pallas as pl
from jax.experimental.pallas import tpu as pltpu


def _cdiv(a: int, b: int) -> int:
    return (a + b - 1) // b


def _mlp_kernel(x_ref, w1_ref, b1_ref, w2_ref, b2_ref, o_ref):
    xb = x_ref[...].astype(jnp.bfloat16)
    w1b = w1_ref[...].astype(jnp.bfloat16)
    w2b = w2_ref[...].astype(jnp.bfloat16)
    h = jnp.dot(xb, w1b, preferred_element_type=jnp.float32)
    h = jnp.maximum(h + b1_ref[...], 0.0).astype(jnp.bfloat16)
    y = jnp.dot(h, w2b, preferred_element_type=jnp.float32)
    o_ref[...] = (y + b2_ref[...]).astype(o_ref.dtype)


@jax.jit
def kernel(x, w1_t, b1_r, w2_t, b2_r):
    b, n_feature = x.shape
    n_hidden, n_output = w2_t.shape

    w1_bf = w1_t
    w2_bf = w2_t
    b1_f = b1_r.astype(jnp.float32)
    b2_f = b2_r.astype(jnp.float32)

    tb = min(1024, max(8, _cdiv(b, 8) * 8))
    nb = _cdiv(b, tb)
    b_pad = nb * tb
    if b_pad != b:
        x = jnp.pad(x, ((0, b_pad - b), (0, 0)))

    out = pl.pallas_call(
        _mlp_kernel,
        out_shape=jax.ShapeDtypeStruct((b_pad, n_output), x.dtype),
        grid=(nb,),
        in_specs=[
            pl.BlockSpec((tb, n_feature), lambda i: (i, 0)),
            pl.BlockSpec((n_feature, n_hidden), lambda i: (0, 0)),
            pl.BlockSpec((1, n_hidden), lambda i: (0, 0)),
            pl.BlockSpec((n_hidden, n_output), lambda i: (0, 0)),
            pl.BlockSpec((1, n_output), lambda i: (0, 0)),
        ],
        out_specs=pl.BlockSpec((tb, n_output), lambda i: (i, 0)),
        compiler_params=pltpu.CompilerParams(
            dimension_semantics=("parallel",),
            vmem_limit_bytes=int(64 * 1024 * 1024 * 0.92)),
    )(x, w1_bf, b1_f, w2_bf, b2_f)

    if b_pad != b:
        out = out[:b]
    return out
